# baseline (speedup 1.0000x reference)
import functools

import jax
import jax.numpy as jnp
from jax import lax
from jax.experimental import pallas as pl
from jax.experimental.pallas import tpu as pltpu
from jax.experimental.pallas import tpu_sc as plsc

_NB = 16


@functools.lru_cache(maxsize=None)
def _make_gather(batch, seq, table_rows, d):
    mesh = plsc.VectorSubcoreMesh(core_axis_name="c", subcore_axis_name="s")
    nc = mesh.num_cores
    num_workers = nc * mesh.num_subcores
    b_per_w = batch // num_workers
    nsuper = b_per_w // _NB

    @functools.partial(
        pl.kernel,
        mesh=mesh,
        out_type=jax.ShapeDtypeStruct((batch, seq, d), jnp.float32),
        compiler_params=pltpu.CompilerParams(use_tc_tiling_on_sc=False),
        scratch_types=[
            pltpu.VMEM((b_per_w, seq), jnp.int32),
            pltpu.VMEM((2, _NB, seq, d), jnp.float32),
            pltpu.SemaphoreType.DMA,
            pltpu.SemaphoreType.DMA,
            pltpu.SemaphoreType.DMA,
            pltpu.SemaphoreType.DMA,
        ],
    )
    def gather(ids_hbm, table_hbm, out_hbm, idx_v, sbuf, sg0, sg1, sw0, sw1):
        wid = lax.axis_index("s") * nc + lax.axis_index("c")
        base_b = wid * b_per_w
        sgs = (sg0, sg1)
        sws = (sw0, sw1)

        pltpu.sync_copy(ids_hbm.at[pl.ds(base_b, b_per_w)], idx_v)

        def out_slice(t):
            return out_hbm.at[pl.ds(base_b + t * _NB, _NB)]

        def fire_gathers(t, buf):
            for g in range(_NB):
                pltpu.async_copy(
                    table_hbm.at[idx_v.at[t * _NB + g]],
                    sbuf.at[buf, g],
                    sgs[buf],
                )

        def wait_gathers(t, buf):
            pltpu.make_async_copy(out_slice(t), sbuf.at[buf], sgs[buf]).wait()

        def wait_write(t, buf):
            pltpu.make_async_copy(out_slice(t), sbuf.at[buf], sws[buf]).wait()

        fire_gathers(0, 0)

        def step(i, carry):
            for s in range(2):
                t = i * 2 + s
                wait_gathers(t, s)
                pltpu.async_copy(sbuf.at[s], out_slice(t), sws[s])
                b = 1 - s
                if s == 0:
                    @pl.when(i >= 1)
                    def _():
                        wait_write(t - 1, b)

                    fire_gathers(t + 1, b)
                else:
                    wait_write(t - 1, b)

                    @pl.when(t + 1 < nsuper)
                    def _():
                        fire_gathers(t + 1, b)

            return carry

        lax.fori_loop(0, nsuper // 2, step, 0)
        wait_write(nsuper - 1, 1)

    return gather


def kernel(ids, table):
    batch, seq = ids.shape
    d = table.shape[1]
    ids32 = ids.astype(jnp.int32)

    info = plsc.get_sparse_core_info()
    num_workers = info.num_cores * info.num_subcores
    assert batch % (num_workers * 2 * _NB) == 0 and seq <= 128
    return _make_gather(batch, seq, table.shape[0], d)(ids32, table)


# speedup vs baseline: 1.6426x; 1.6426x over previous
import functools

import jax
import jax.numpy as jnp
from jax import lax
from jax.experimental import pallas as pl
from jax.experimental.pallas import tpu as pltpu
from jax.experimental.pallas import tpu_sc as plsc

_L = 16
_UNROLL = 8


@functools.lru_cache(maxsize=None)
def _make_gather(seq, batch, d, vocab):
    mesh = plsc.VectorSubcoreMesh(core_axis_name="c", subcore_axis_name="s")
    nc = mesh.num_cores
    num_workers = nc * mesh.num_subcores
    d_per_w = d // num_workers
    steps = batch // (_L * _UNROLL)

    @functools.partial(
        pl.kernel,
        mesh=mesh,
        out_type=jax.ShapeDtypeStruct((seq, d, batch), jnp.float32),
        compiler_params=pltpu.CompilerParams(
            use_tc_tiling_on_sc=True, needs_layout_passes=False
        ),
        scratch_types=[
            pltpu.VMEM((vocab,), jnp.float32),
            pltpu.VMEM((batch,), jnp.int32),
            pltpu.VMEM((batch,), jnp.int32),
            pltpu.VMEM((batch,), jnp.float32),
            pltpu.VMEM((batch,), jnp.float32),
            pltpu.SemaphoreType.DMA,
            pltpu.SemaphoreType.DMA,
            pltpu.SemaphoreType.DMA,
            pltpu.SemaphoreType.DMA,
        ],
    )
    def gather(ids_hbm, table_hbm, out_hbm, row_v, ib0, ib1, ob0, ob1,
               si0, si1, so0, so1):
        wid = lax.axis_index("s") * nc + lax.axis_index("c")
        ibs = (ib0, ib1)
        obs = (ob0, ob1)
        sis = (si0, si1)
        sos = (so0, so1)

        def compute(ib, ob):
            def step(k, carry):
                for u in range(_UNROLL):
                    off = (k * _UNROLL + u) * _L
                    idx = ib[pl.ds(off, _L)]
                    ob[pl.ds(off, _L)] = plsc.load_gather(row_v, [idx])
                return carry

            lax.fori_loop(0, steps, step, 0)

        def run_feature(dd):
            pltpu.sync_copy(table_hbm.at[dd], row_v)
            pltpu.async_copy(ids_hbm.at[0], ibs[0], sis[0])

            def s_iter(i, carry):
                for b in range(2):
                    s = i * 2 + b
                    nb = 1 - b
                    pltpu.make_async_copy(ids_hbm.at[s], ibs[b], sis[b]).wait()
                    if b == 0:
                        pltpu.async_copy(ids_hbm.at[s + 1], ibs[nb], sis[nb])
                    else:
                        @pl.when(s + 1 < seq)
                        def _():
                            pltpu.async_copy(ids_hbm.at[s + 1], ibs[nb], sis[nb])

                    @pl.when(i >= 1)
                    def _():
                        pltpu.make_async_copy(
                            obs[b], out_hbm.at[s - 2, dd], sos[b]
                        ).wait()

                    compute(ibs[b], obs[b])
                    pltpu.async_copy(obs[b], out_hbm.at[s, dd], sos[b])
                return carry

            lax.fori_loop(0, seq // 2, s_iter, 0)
            for b in range(2):
                pltpu.make_async_copy(obs[b], out_hbm.at[seq - 2 + b, dd],
                                      sos[b]).wait()

        for dp in range(d_per_w):
            run_feature(wid + num_workers * dp)

    return gather


def kernel(ids, table):
    batch, seq = ids.shape
    vocab, d = table.shape
    ids_t = jnp.transpose(ids.astype(jnp.int32))
    table_t = jnp.transpose(table)

    info = plsc.get_sparse_core_info()
    num_workers = info.num_cores * info.num_subcores
    assert d % num_workers == 0 and seq % 2 == 0
    assert batch % (_L * _UNROLL) == 0
    out_t = _make_gather(seq, batch, d, vocab)(ids_t, table_t)
    return jnp.transpose(out_t, (2, 0, 1))


# speedup vs baseline: 1.8456x; 1.1236x over previous
import functools

import jax
import jax.numpy as jnp
from jax import lax
from jax.experimental import pallas as pl
from jax.experimental.pallas import tpu as pltpu
from jax.experimental.pallas import tpu_sc as plsc

_L = 16
_UNROLL = 8


@functools.lru_cache(maxsize=None)
def _make_gather(seq, batch, d, vocab):
    mesh = plsc.VectorSubcoreMesh(core_axis_name="c", subcore_axis_name="s")
    nc = mesh.num_cores
    num_workers = nc * mesh.num_subcores
    d_per_w = d // num_workers
    steps = batch // (_L * _UNROLL)

    @functools.partial(
        pl.kernel,
        mesh=mesh,
        out_type=jax.ShapeDtypeStruct((seq, d, batch), jnp.float32),
        compiler_params=pltpu.CompilerParams(
            use_tc_tiling_on_sc=True, needs_layout_passes=False
        ),
        scratch_types=[
            pltpu.VMEM((vocab,), jnp.float32),
            pltpu.VMEM((batch,), jnp.int32),
            pltpu.VMEM((batch,), jnp.int32),
            pltpu.VMEM((batch,), jnp.float32),
            pltpu.VMEM((batch,), jnp.float32),
            pltpu.SemaphoreType.DMA,
            pltpu.SemaphoreType.DMA,
            pltpu.SemaphoreType.DMA,
            pltpu.SemaphoreType.DMA,
        ],
    )
    def gather(ids_hbm, table_hbm, out_hbm, row_v, ib0, ib1, ob0, ob1,
               si0, si1, so0, so1):
        wid = lax.axis_index("s") * nc + lax.axis_index("c")
        ibs = (ib0, ib1)
        obs = (ob0, ob1)
        sis = (si0, si1)
        sos = (so0, so1)

        def compute(ib, ob):
            @plsc.parallel_loop(0, batch, step=_L, unroll=_UNROLL)
            def _(off):
                idx = ib[pl.ds(off, _L)]
                ob[pl.ds(off, _L)] = plsc.load_gather(row_v, [idx])

        def run_feature(dd):
            pltpu.sync_copy(table_hbm.at[dd], row_v)
            pltpu.async_copy(ids_hbm.at[0], ibs[0], sis[0])

            def s_iter(i, carry):
                for b in range(2):
                    s = i * 2 + b
                    nb = 1 - b
                    pltpu.make_async_copy(ids_hbm.at[s], ibs[b], sis[b]).wait()
                    if b == 0:
                        pltpu.async_copy(ids_hbm.at[s + 1], ibs[nb], sis[nb])
                    else:
                        @pl.when(s + 1 < seq)
                        def _():
                            pltpu.async_copy(ids_hbm.at[s + 1], ibs[nb], sis[nb])

                    @pl.when(i >= 1)
                    def _():
                        pltpu.make_async_copy(
                            obs[b], out_hbm.at[s - 2, dd], sos[b]
                        ).wait()

                    compute(ibs[b], obs[b])
                    pltpu.async_copy(obs[b], out_hbm.at[s, dd], sos[b])
                return carry

            lax.fori_loop(0, seq // 2, s_iter, 0)
            for b in range(2):
                pltpu.make_async_copy(obs[b], out_hbm.at[seq - 2 + b, dd],
                                      sos[b]).wait()

        for dp in range(d_per_w):
            run_feature(wid + num_workers * dp)

    return gather


def kernel(ids, table):
    batch, seq = ids.shape
    vocab, d = table.shape
    ids_t = jnp.transpose(ids.astype(jnp.int32))
    table_t = jnp.transpose(table)

    info = plsc.get_sparse_core_info()
    num_workers = info.num_cores * info.num_subcores
    assert d % num_workers == 0 and seq % 2 == 0
    assert batch % (_L * _UNROLL) == 0
    out_t = _make_gather(seq, batch, d, vocab)(ids_t, table_t)
    return jnp.transpose(out_t, (2, 0, 1))
